# baseline (speedup 1.0000x reference)
import functools

import jax
import jax.numpy as jnp
from jax import lax
from jax.experimental import pallas as pl
from jax.experimental.pallas import tpu as pltpu
from jax.experimental.pallas import tpu_sc as plsc

_N = 10000
_E = 320000
_D = 128
_ALPHA = 0.9

_NC = 2
_NS = 16
_NW = _NC * _NS
_CH = 64
_NCH = 160
_PER_W = _NCH * _CH
_E_PAD = _NW * _PER_W
_N_PAD = 10240
_RPW = _N_PAD // _NS
_NCHH = _NCH // 2


def _sc_body(x_hbm, src_hbm, dst_hbm, val_hbm, out_hbm,
             agg_sh, src_v, dst_v, val_v, rows0, rows1, gsem0, gsem1):
    c = lax.axis_index("c")
    s = lax.axis_index("s")
    w = c * _NS + s

    rows = (rows0, rows1)
    gsem = (gsem0, gsem1)

    def zset(i, carry):
        z = jnp.zeros((16,), jnp.float32)
        for f in range(_D // 16):
            rows0[i, pl.ds(f * 16, 16)] = z
        return carry

    lax.fori_loop(0, _CH, zset, 0)

    def zdma(k, carry):
        pltpu.sync_copy(rows0, agg_sh.at[pl.ds(s * _RPW + k * _CH, _CH)])
        return carry

    lax.fori_loop(0, _RPW // _CH, zdma, 0)

    plsc.subcore_barrier()

    def issue(j, b):
        pltpu.async_copy(x_hbm.at[src_v.at[j]], rows[b], gsem[b])

    def wait(j, b):
        pltpu.make_async_copy(x_hbm.at[src_v.at[j]], rows[b], gsem[b]).wait()

    def scale(j, b):
        rv = rows[b]

        def group(g, inner):
            vv16 = val_v[j, pl.ds(g * 16, 16)]
            for e16 in range(16):
                bc = jnp.take_along_axis(
                    vv16, jnp.full((16,), e16, jnp.int32), axis=0)
                e = g * 16 + e16
                for f in range(_D // 16):
                    sl = pl.ds(f * 16, 16)
                    rv[e, sl] = rv[e, sl] * bc
            return inner

        lax.fori_loop(0, _CH // 16, group, 0)

    def step(j, b, prefetch):
        if prefetch:
            issue(j + 1, 1 - b)
        wait(j, b)
        scale(j, b)
        pltpu.sync_copy(rows[b], agg_sh.at[dst_v.at[j]], add=True)

    def half(h, carry):
        hs = pl.ds(h * _NCHH, _NCHH)
        pltpu.sync_copy(src_hbm.at[w, hs], src_v)
        pltpu.sync_copy(dst_hbm.at[w, hs], dst_v)
        pltpu.sync_copy(val_hbm.at[w, hs], val_v)
        issue(0, 0)

        def pair(m, inner):
            j0 = 2 * m
            step(j0, 0, True)
            step(j0 + 1, 1, True)
            return inner

        lax.fori_loop(0, _NCHH // 2 - 1, pair, 0)
        step(_NCHH - 2, 0, True)
        step(_NCHH - 1, 1, False)
        return carry

    lax.fori_loop(0, _NCH // _NCHH, half, 0)

    plsc.subcore_barrier()

    pltpu.sync_copy(agg_sh.at[pl.ds(s * _RPW, _RPW)],
                    out_hbm.at[c, pl.ds(s * _RPW, _RPW)])


_sc_gather_scatter = functools.partial(
    pl.kernel,
    out_type=jax.ShapeDtypeStruct((_NC, _N_PAD, _D), jnp.float32),
    mesh=plsc.VectorSubcoreMesh(core_axis_name="c", subcore_axis_name="s"),
    scratch_types=[
        pltpu.VMEM_SHARED((_N_PAD, _D), jnp.float32),
        pltpu.VMEM((_NCHH, _CH), jnp.int32),
        pltpu.VMEM((_NCHH, _CH), jnp.int32),
        pltpu.VMEM((_NCHH, _CH), jnp.float32),
        pltpu.VMEM((_CH, _D), jnp.float32),
        pltpu.VMEM((_CH, _D), jnp.float32),
        pltpu.SemaphoreType.DMA,
        pltpu.SemaphoreType.DMA,
    ],
)(_sc_body)


_BLK = 400


def _tc_body(p_ref, xi_ref, wi_ref, wx_ref, o_ref):
    agg = p_ref[0] + p_ref[1]
    h = _ALPHA * agg + (1.0 - _ALPHA) * jnp.dot(
        xi_ref[...], wi_ref[...], preferred_element_type=jnp.float32)
    o_ref[...] = jnp.maximum(
        jnp.dot(h, wx_ref[...], preferred_element_type=jnp.float32), 0.0)


def _tc_dense(partials, xi, wi, wx):
    nblk = _N // _BLK
    return pl.pallas_call(
        _tc_body,
        out_shape=jax.ShapeDtypeStruct((_N, _D), jnp.float32),
        grid=(nblk,),
        in_specs=[
            pl.BlockSpec((2, _BLK, _D), lambda i: (0, i, 0)),
            pl.BlockSpec((_BLK, 8), lambda i: (i, 0)),
            pl.BlockSpec((8, _D), lambda i: (0, 0)),
            pl.BlockSpec((_D, _D), lambda i: (0, 0)),
        ],
        out_specs=pl.BlockSpec((_BLK, _D), lambda i: (i, 0)),
    )(partials, xi, wi, wx)


def kernel(x, x_initial, edge_index, adj_values, w_init, w_x):
    dst = edge_index[0]
    src = edge_index[1]
    pad = _E_PAD - _E
    zi = jnp.zeros((pad,), jnp.int32)
    srcp = jnp.concatenate([src, zi]).reshape(_NW, _NCH, _CH)
    dstp = jnp.concatenate([dst, zi]).reshape(_NW, _NCH, _CH)
    valp = jnp.concatenate(
        [adj_values, jnp.zeros((pad,), jnp.float32)]).reshape(_NW, _NCH, _CH)

    partials = _sc_gather_scatter(x, srcp, dstp, valp)

    xi = jnp.pad(x_initial, ((0, 0), (0, 5)))
    wi = jnp.pad(w_init, ((0, 5), (0, 0)))
    return _tc_dense(partials, xi, wi, w_x)
